# Initial kernel scaffold; baseline (speedup 1.0000x reference)
import jax
import jax.numpy as jnp
from jax.experimental import pallas as pl


def kernel(X, f, edge_weight, K, K_features, edge_index):
    raise NotImplementedError("write your pallas kernel here")


# speedup vs baseline: 3.6027x; 3.6027x over previous
import functools
import math

import jax
import jax.numpy as jnp
from jax import lax
from jax.experimental import pallas as pl
from jax.experimental.pallas import tpu as pltpu
from jax.experimental.pallas import tpu_sc as plsc

NLAYERS = 4
NFIX = 2
N = 10000
E = 320000
C = 128
L = 16
NC = 2
NS = 16
NW = NC * NS
EPW = E // NW
CH = 80
NCH = EPW // CH
NP = 10112
RPS = NP // NS


def _sc_edge_body(dz_hbm, src_hbm, dst_hbm, ewrep_hbm, zrows_hbm, out_hbm,
                  srcT, dstT, srcv, gsv, gdv, ewc, av, bv, aggsh,
                  sem_a, sem_b):
    cid = lax.axis_index("c")
    sid = lax.axis_index("s")
    wid = cid * NS + sid
    ebase = wid * EPW
    rbase = sid * RPS

    pltpu.sync_copy(src_hbm.at[pl.ds(ebase, EPW)], srcT)
    pltpu.sync_copy(dst_hbm.at[pl.ds(ebase, EPW)], dstT)

    for i in range(NLAYERS):
        pltpu.sync_copy(zrows_hbm, aggsh.at[pl.ds(rbase, RPS)])
        plsc.subcore_barrier()

        def chunk_body(t, _, i=i):
            eb = t * CH
            for j in range(CH // L):
                sl = pl.ds(j * L, L)
                s16 = srcT[pl.ds(eb + j * L, L)]
                d16 = dstT[pl.ds(eb + j * L, L)]
                srcv[sl] = s16
                gsv[sl] = s16 + i * N
                gdv[sl] = d16 + i * N
            pltpu.sync_copy(ewrep_hbm.at[pl.ds((ebase + eb) * L, CH * L)], ewc)
            ca = pltpu.async_copy(dz_hbm.at[gsv], av, sem_a)
            cb = pltpu.async_copy(dz_hbm.at[gdv], bv, sem_b)
            ca.wait()
            cb.wait()

            def edge_body(e, _):
                ews = ewc[pl.ds(e * L, L)]
                for k in range(C // L):
                    sl = pl.ds(k * L, L)
                    a16 = av[e, sl]
                    b16 = bv[e, sl]
                    mg = (b16 - a16) * ews
                    den = 1.0 + jnp.exp(mg)
                    bv[e, sl] = -(mg / den)
                return 0

            lax.fori_loop(0, CH, edge_body, 0)
            pltpu.sync_copy(bv, aggsh.at[srcv], add=True)
            return 0

        lax.fori_loop(0, NCH, chunk_body, 0)
        plsc.subcore_barrier()
        pltpu.sync_copy(aggsh.at[pl.ds(rbase, RPS)],
                        out_hbm.at[cid, i, pl.ds(rbase, RPS)])


_sc_edge = functools.partial(
    pl.kernel,
    out_type=jax.ShapeDtypeStruct((NC, NLAYERS, NP, C), jnp.float32),
    mesh=plsc.VectorSubcoreMesh(core_axis_name="c", subcore_axis_name="s",
                                num_cores=NC, num_subcores=NS),
    scratch_types=[
        pltpu.VMEM((EPW,), jnp.int32),
        pltpu.VMEM((EPW,), jnp.int32),
        pltpu.VMEM((CH,), jnp.int32),
        pltpu.VMEM((CH,), jnp.int32),
        pltpu.VMEM((CH,), jnp.int32),
        pltpu.VMEM((CH * L,), jnp.float32),
        pltpu.VMEM((CH, C), jnp.float32),
        pltpu.VMEM((CH, C), jnp.float32),
        pltpu.VMEM_SHARED((NP, C), jnp.float32),
        pltpu.SemaphoreType.DMA,
        pltpu.SemaphoreType.DMA,
    ],
)(_sc_edge_body)


@jax.jit
def _edge_op(dZ, src, dst, ewrep, zrows):
    parts = _sc_edge(dZ.reshape(NLAYERS * N, C), src, dst, ewrep, zrows)
    return parts[0, :, :N] + parts[1, :, :N]


def _silu(x):
    return x * jax.nn.sigmoid(x)


@jax.jit
def _dense_in(Z, f, Kf, K):
    Zc = [_silu(jnp.concatenate([Z[i], f], -1) @ Kf[i]) for i in range(NLAYERS)]
    return jnp.stack([Zc[i] @ K[i] for i in range(NLAYERS)])


@jax.jit
def _dense_out(agg, K, X):
    Ys = [-jnp.einsum('nc,dc->nd', agg[i], K[i]) for i in range(NLAYERS)]
    Ys[-1] = Ys[-1] + X
    Y = [None] * NLAYERS
    Y[0] = math.sqrt(0.5) * Ys[0]
    for i in range(1, NLAYERS):
        a = math.sqrt((i + 1) / (i + 2))
        b = math.sqrt(i / (i + 1))
        Y[i] = a * (b * Y[i - 1] + Ys[i])
    W = [None] * NLAYERS
    W[NLAYERS - 1] = math.sqrt(NLAYERS / (NLAYERS + 1)) * Y[NLAYERS - 1]
    for i in range(NLAYERS - 2, -1, -1):
        a = math.sqrt((i + 1) / (i + 2))
        W[i] = a * (a * W[i + 1] + Y[i])
    return jnp.stack(W)


def kernel(X, f, edge_weight, K, K_features, edge_index):
    src = edge_index[0]
    dst = edge_index[1]
    zrows = jnp.zeros((RPS, C), jnp.float32)
    ewrep = jnp.repeat(edge_weight[:, None], L, axis=1).reshape(E * L)
    Z = jnp.zeros((NLAYERS, N, C), dtype=jnp.float32)
    for _ in range(NFIX):
        dZ = _dense_in(Z, f, K_features, K)
        agg = _edge_op(dZ, src, dst, ewrep, zrows)
        Z = _dense_out(agg, K, X)
    return (Z[-1], Z)


# speedup vs baseline: 3.6190x; 1.0045x over previous
import functools
import math

import jax
import jax.numpy as jnp
from jax import lax
from jax.experimental import pallas as pl
from jax.experimental.pallas import tpu as pltpu
from jax.experimental.pallas import tpu_sc as plsc

NLAYERS = 4
NFIX = 2
N = 10000
E = 320000
C = 128
L = 16
NC = 2
NS = 16
NW = NC * NS
EPW = E // NW
CH = 80
NCH = EPW // CH
NP = 10112
RPS = NP // NS


def _sc_edge_body(dz_hbm, src_hbm, dst_hbm, ewrep_hbm, zrows_hbm, out_hbm,
                  srcT, dstT, srcv, gsv, gdv, ewc, av, bv, aggsh,
                  sem_a, sem_b):
    cid = lax.axis_index("c")
    sid = lax.axis_index("s")
    wid = cid * NS + sid
    ebase = wid * EPW
    rbase = sid * RPS

    pltpu.sync_copy(src_hbm.at[pl.ds(ebase, EPW)], srcT)
    pltpu.sync_copy(dst_hbm.at[pl.ds(ebase, EPW)], dstT)

    for i in range(NLAYERS):
        pltpu.sync_copy(zrows_hbm, aggsh.at[pl.ds(rbase, RPS)])
        plsc.subcore_barrier()

        def chunk_body(t, _, i=i):
            eb = t * CH
            for j in range(CH // L):
                sl = pl.ds(j * L, L)
                s16 = srcT[pl.ds(eb + j * L, L)]
                d16 = dstT[pl.ds(eb + j * L, L)]
                srcv[sl] = s16
                gsv[sl] = s16 + i * N
                gdv[sl] = d16 + i * N
            pltpu.sync_copy(ewrep_hbm.at[pl.ds((ebase + eb) * L, CH * L)], ewc)
            ca = pltpu.async_copy(dz_hbm.at[gsv], av, sem_a)
            cb = pltpu.async_copy(dz_hbm.at[gdv], bv, sem_b)
            ca.wait()
            cb.wait()

            def edge_body(e, _):
                ews = ewc[pl.ds(e * L, L)]
                for k in range(C // L):
                    sl = pl.ds(k * L, L)
                    a16 = av[e, sl]
                    b16 = bv[e, sl]
                    mg = (b16 - a16) * ews
                    den = 1.0 + jnp.exp(mg)
                    bv[e, sl] = -(mg / den)
                return 0

            lax.fori_loop(0, CH, edge_body, 0)
            pltpu.sync_copy(bv, aggsh.at[srcv], add=True)
            return 0

        lax.fori_loop(0, NCH, chunk_body, 0)
        plsc.subcore_barrier()
        pltpu.sync_copy(aggsh.at[pl.ds(rbase, RPS)],
                        out_hbm.at[cid, i, pl.ds(rbase, RPS)])


_sc_edge = functools.partial(
    pl.kernel,
    out_type=jax.ShapeDtypeStruct((NC, NLAYERS, NP, C), jnp.float32),
    mesh=plsc.VectorSubcoreMesh(core_axis_name="c", subcore_axis_name="s",
                                num_cores=NC, num_subcores=NS),
    scratch_types=[
        pltpu.VMEM((EPW,), jnp.int32),
        pltpu.VMEM((EPW,), jnp.int32),
        pltpu.VMEM((CH,), jnp.int32),
        pltpu.VMEM((CH,), jnp.int32),
        pltpu.VMEM((CH,), jnp.int32),
        pltpu.VMEM((CH * L,), jnp.float32),
        pltpu.VMEM((CH, C), jnp.float32),
        pltpu.VMEM((CH, C), jnp.float32),
        pltpu.VMEM_SHARED((NP, C), jnp.float32),
        pltpu.SemaphoreType.DMA,
        pltpu.SemaphoreType.DMA,
    ],
)(_sc_edge_body)


@jax.jit
def _edge_op(dZ, src, dst, ewrep, zrows):
    return _sc_edge(dZ.reshape(NLAYERS * N, C), src, dst, ewrep, zrows)


def _silu(x):
    return x * jax.nn.sigmoid(x)


BN = 1000


def _dense_in_body(z_ref, f_ref, kf_ref, k_ref, dz_ref):
    z = z_ref[0]
    fb = f_ref[...]
    kf = kf_ref[0]
    zc = (jnp.dot(z, kf[:C], preferred_element_type=jnp.float32)
          + jnp.dot(fb, kf[C:], preferred_element_type=jnp.float32))
    zc = _silu(zc)
    dz_ref[0] = jnp.dot(zc, k_ref[0], preferred_element_type=jnp.float32)


@jax.jit
def _dense_in(Z, f, Kf, K):
    return pl.pallas_call(
        _dense_in_body,
        grid=(NLAYERS, N // BN),
        in_specs=[
            pl.BlockSpec((1, BN, C), lambda i, j: (i, j, 0)),
            pl.BlockSpec((BN, C), lambda i, j: (j, 0)),
            pl.BlockSpec((1, 2 * C, C), lambda i, j: (i, 0, 0)),
            pl.BlockSpec((1, C, C), lambda i, j: (i, 0, 0)),
        ],
        out_specs=pl.BlockSpec((1, BN, C), lambda i, j: (i, j, 0)),
        out_shape=jax.ShapeDtypeStruct((NLAYERS, N, C), jnp.float32),
    )(Z, f, Kf, K)


def _dense_out_body(pa_ref, pb_ref, k_ref, x_ref, z_ref):
    Ys = []
    for i in range(NLAYERS):
        agg = pa_ref[0, i] + pb_ref[0, i]
        y = -lax.dot_general(agg, k_ref[i], (((1,), (1,)), ((), ())),
                             preferred_element_type=jnp.float32)
        if i == NLAYERS - 1:
            y = y + x_ref[...]
        Ys.append(y)
    Yt = [None] * NLAYERS
    Yt[0] = math.sqrt(0.5) * Ys[0]
    for i in range(1, NLAYERS):
        a = math.sqrt((i + 1) / (i + 2))
        b = math.sqrt(i / (i + 1))
        Yt[i] = a * (b * Yt[i - 1] + Ys[i])
    W = [None] * NLAYERS
    W[NLAYERS - 1] = math.sqrt(NLAYERS / (NLAYERS + 1)) * Yt[NLAYERS - 1]
    for i in range(NLAYERS - 2, -1, -1):
        a = math.sqrt((i + 1) / (i + 2))
        W[i] = a * (a * W[i + 1] + Yt[i])
    for i in range(NLAYERS):
        z_ref[i] = W[i]


@jax.jit
def _dense_out(parts, K, X):
    return pl.pallas_call(
        _dense_out_body,
        grid=(N // BN,),
        in_specs=[
            pl.BlockSpec((1, NLAYERS, BN, C), lambda j: (0, 0, j, 0)),
            pl.BlockSpec((1, NLAYERS, BN, C), lambda j: (1, 0, j, 0)),
            pl.BlockSpec((NLAYERS, C, C), lambda j: (0, 0, 0)),
            pl.BlockSpec((BN, C), lambda j: (j, 0)),
        ],
        out_specs=pl.BlockSpec((NLAYERS, BN, C), lambda j: (0, j, 0)),
        out_shape=jax.ShapeDtypeStruct((NLAYERS, N, C), jnp.float32),
    )(parts, parts, K, X)


def kernel(X, f, edge_weight, K, K_features, edge_index):
    src = edge_index[0]
    dst = edge_index[1]
    zrows = jnp.zeros((RPS, C), jnp.float32)
    ewrep = jnp.repeat(edge_weight[:, None], L, axis=1).reshape(E * L)
    Z = jnp.zeros((NLAYERS, N, C), dtype=jnp.float32)
    for _ in range(NFIX):
        dZ = _dense_in(Z, f, K_features, K)
        parts = _edge_op(dZ, src, dst, ewrep, zrows)
        Z = _dense_out(parts, K, X)
    return (Z[-1], Z)


# speedup vs baseline: 3.8631x; 1.0674x over previous
import functools
import math

import jax
import jax.numpy as jnp
from jax import lax
from jax.experimental import pallas as pl
from jax.experimental.pallas import tpu as pltpu
from jax.experimental.pallas import tpu_sc as plsc

NLAYERS = 4
NFIX = 2
N = 10000
E = 320000
C = 128
L = 16
NC = 2
NS = 16
NW = NC * NS
EPW = E // NW
CH = 40
NCH = EPW // CH
NSLOT = 4
NP = 10112
RPS = NP // NS


def _sc_edge_body(dz_hbm, ndz_hbm, idx3_hbm, ewn_hbm, zrows_hbm, out_hbm,
                  aggsh,
                  idxb0, ewc0, av0, idxb1, ewc1, av1,
                  idxb2, ewc2, av2, idxb3, ewc3, av3,
                  *sems):
    cid = lax.axis_index("c")
    sid = lax.axis_index("s")
    wid = cid * NS + sid
    ebase = wid * EPW
    cgbase = wid * NCH
    rbase = sid * RPS

    slots = [
        (idxb0, ewc0, av0) + sems[0:5],
        (idxb1, ewc1, av1) + sems[5:10],
        (idxb2, ewc2, av2) + sems[10:15],
        (idxb3, ewc3, av3) + sems[15:20],
    ]

    def fire_idx(c, slot, i):
        idxb, ewc, av, si, se, sa, sb, ss = slot
        pltpu.async_copy(idx3_hbm.at[i, cgbase + c], idxb, si)
        pltpu.async_copy(ewn_hbm.at[pl.ds((ebase + c * CH) * L, CH * L)],
                         ewc, se)

    def wait_scatter(slot):
        idxb, ewc, av, si, se, sa, sb, ss = slot
        pltpu.make_async_copy(av, aggsh.at[idxb.at[0]], ss).wait()

    def g1_stage(c, slot, i):
        idxb, ewc, av, si, se, sa, sb, ss = slot
        pltpu.make_async_copy(idx3_hbm.at[i, cgbase + c], idxb, si).wait()
        pltpu.async_copy(ndz_hbm.at[idxb.at[2]], av, sa)

    def g2_stage(slot):
        idxb, ewc, av, si, se, sa, sb, ss = slot
        pltpu.make_async_copy(ndz_hbm.at[idxb.at[2]], av, sa).wait()
        pltpu.async_copy(dz_hbm.at[idxb.at[1]], av, sb, add=True)

    def compute_stage(c, slot):
        idxb, ewc, av, si, se, sa, sb, ss = slot
        pltpu.make_async_copy(dz_hbm.at[idxb.at[1]], av, sb).wait()
        pltpu.make_async_copy(ewn_hbm.at[pl.ds((ebase + c * CH) * L, CH * L)],
                              ewc, se).wait()

        def edge_body(e, _):
            ews = ewc[pl.ds(e * L, L)]
            for k in range(C // L):
                sl = pl.ds(k * L, L)
                mg = av[e, sl] * ews
                den = 1.0 + jnp.exp(mg)
                av[e, sl] = -(mg / den)
            return 0

        lax.fori_loop(0, CH, edge_body, 0)
        pltpu.async_copy(av, aggsh.at[idxb.at[0]], ss, add=True)

    NBODY = NCH + 3
    NTRIP = (NBODY + NSLOT - 1) // NSLOT

    for i in range(NLAYERS):
        pltpu.sync_copy(zrows_hbm, aggsh.at[pl.ds(rbase, RPS)])
        plsc.subcore_barrier()

        def trip_body(p, _, i=i):
            for bb in range(NSLOT):
                n = p * NSLOT + bb

                @pl.when(jnp.logical_and(n - 3 >= 0, n - 3 < NCH))
                def _(n=n, bb=bb):
                    compute_stage(n - 3, slots[(bb - 3) % NSLOT])

                @pl.when(jnp.logical_and(n - 2 >= 0, n - 2 < NCH))
                def _(n=n, bb=bb):
                    g2_stage(slots[(bb - 2) % NSLOT])

                @pl.when(jnp.logical_and(n - 1 >= 0, n - 1 < NCH))
                def _(n=n, bb=bb, i=i):
                    g1_stage(n - 1, slots[(bb - 1) % NSLOT], i)

                @pl.when(n < NCH)
                def _(n=n, bb=bb, i=i):
                    @pl.when(n >= NSLOT)
                    def _(bb=bb):
                        wait_scatter(slots[bb])
                    fire_idx(n, slots[bb], i)
            return 0

        lax.fori_loop(0, NTRIP, trip_body, 0)
        for bb in range(NSLOT):
            wait_scatter(slots[bb])
        plsc.subcore_barrier()
        pltpu.sync_copy(aggsh.at[pl.ds(rbase, RPS)],
                        out_hbm.at[cid, i, pl.ds(rbase, RPS)])


_sc_edge = functools.partial(
    pl.kernel,
    out_type=jax.ShapeDtypeStruct((NC, NLAYERS, NP, C), jnp.float32),
    mesh=plsc.VectorSubcoreMesh(core_axis_name="c", subcore_axis_name="s",
                                num_cores=NC, num_subcores=NS),
    scratch_types=(
        [pltpu.VMEM_SHARED((NP, C), jnp.float32)]
        + [pltpu.VMEM((3, CH), jnp.int32) if k % 3 == 0
           else (pltpu.VMEM((CH * L,), jnp.float32) if k % 3 == 1
                 else pltpu.VMEM((CH, C), jnp.float32))
           for k in range(3 * NSLOT)]
        + [pltpu.SemaphoreType.DMA] * (5 * NSLOT)
    ),
)(_sc_edge_body)


@jax.jit
def _edge_op(dZ, nDZ, idx3, ewn, zrows):
    return _sc_edge(dZ, nDZ, idx3, ewn, zrows)


def _silu(x):
    return x * jax.nn.sigmoid(x)


BN = 1000


def _dense_in_body(z_ref, f_ref, kf_ref, k_ref, dz_ref, ndz_ref):
    z = z_ref[0]
    fb = f_ref[...]
    kf = kf_ref[0]
    zc = (jnp.dot(z, kf[:C], preferred_element_type=jnp.float32)
          + jnp.dot(fb, kf[C:], preferred_element_type=jnp.float32))
    zc = _silu(zc)
    dz = jnp.dot(zc, k_ref[0], preferred_element_type=jnp.float32)
    dz_ref[0] = dz
    ndz_ref[0] = -dz


@jax.jit
def _dense_in(Z, f, Kf, K):
    return pl.pallas_call(
        _dense_in_body,
        grid=(NLAYERS, N // BN),
        in_specs=[
            pl.BlockSpec((1, BN, C), lambda i, j: (i, j, 0)),
            pl.BlockSpec((BN, C), lambda i, j: (j, 0)),
            pl.BlockSpec((1, 2 * C, C), lambda i, j: (i, 0, 0)),
            pl.BlockSpec((1, C, C), lambda i, j: (i, 0, 0)),
        ],
        out_specs=[
            pl.BlockSpec((1, BN, C), lambda i, j: (i, j, 0)),
            pl.BlockSpec((1, BN, C), lambda i, j: (i, j, 0)),
        ],
        out_shape=[
            jax.ShapeDtypeStruct((NLAYERS, N, C), jnp.float32),
            jax.ShapeDtypeStruct((NLAYERS, N, C), jnp.float32),
        ],
    )(Z, f, Kf, K)


def _dense_out_body(pa_ref, pb_ref, k_ref, x_ref, z_ref):
    Ys = []
    for i in range(NLAYERS):
        agg = pa_ref[0, i] + pb_ref[0, i]
        y = -lax.dot_general(agg, k_ref[i], (((1,), (1,)), ((), ())),
                             preferred_element_type=jnp.float32)
        if i == NLAYERS - 1:
            y = y + x_ref[...]
        Ys.append(y)
    Yt = [None] * NLAYERS
    Yt[0] = math.sqrt(0.5) * Ys[0]
    for i in range(1, NLAYERS):
        a = math.sqrt((i + 1) / (i + 2))
        b = math.sqrt(i / (i + 1))
        Yt[i] = a * (b * Yt[i - 1] + Ys[i])
    W = [None] * NLAYERS
    W[NLAYERS - 1] = math.sqrt(NLAYERS / (NLAYERS + 1)) * Yt[NLAYERS - 1]
    for i in range(NLAYERS - 2, -1, -1):
        a = math.sqrt((i + 1) / (i + 2))
        W[i] = a * (a * W[i + 1] + Yt[i])
    for i in range(NLAYERS):
        z_ref[i] = W[i]


@jax.jit
def _dense_out(parts, K, X):
    return pl.pallas_call(
        _dense_out_body,
        grid=(N // BN,),
        in_specs=[
            pl.BlockSpec((1, NLAYERS, BN, C), lambda j: (0, 0, j, 0)),
            pl.BlockSpec((1, NLAYERS, BN, C), lambda j: (1, 0, j, 0)),
            pl.BlockSpec((NLAYERS, C, C), lambda j: (0, 0, 0)),
            pl.BlockSpec((BN, C), lambda j: (j, 0)),
        ],
        out_specs=pl.BlockSpec((NLAYERS, BN, C), lambda j: (0, j, 0)),
        out_shape=jax.ShapeDtypeStruct((NLAYERS, N, C), jnp.float32),
    )(parts, parts, K, X)


def kernel(X, f, edge_weight, K, K_features, edge_index):
    src = edge_index[0]
    dst = edge_index[1]
    zrows = jnp.zeros((RPS, C), jnp.float32)
    ewn = jnp.repeat(-edge_weight[:, None], L, axis=1).reshape(E * L)
    srcr = src.reshape(E // CH, CH)
    dstr = dst.reshape(E // CH, CH)
    idx3 = jnp.stack([
        jnp.stack([srcr, srcr + i * N, dstr + i * N], axis=1)
        for i in range(NLAYERS)
    ])
    Z = jnp.zeros((NLAYERS, N, C), dtype=jnp.float32)
    for _ in range(NFIX):
        dZ, nDZ = _dense_in(Z, f, K_features, K)
        parts = _edge_op(dZ.reshape(NLAYERS * N, C),
                         nDZ.reshape(NLAYERS * N, C), idx3, ewn, zrows)
        Z = _dense_out(parts, K, X)
    return (Z[-1], Z)
